# baseline (speedup 1.0000x reference)
import jax
import jax.numpy as jnp
from jax.experimental import pallas as pl

BATCH = 32
MAX_SEQ_LEN = 4096
KV_HEAD_DIM = 128
PREFILL_LEN = 2048

SEQ_BLOCK = 256
N_BLOCKS = MAX_SEQ_LEN // SEQ_BLOCK
N_PREFILL_BLOCKS = PREFILL_LEN // SEQ_BLOCK


def _body(kx_ref, vx_ref, k_out, v_out):
    j = pl.program_id(0)

    @pl.when(j % 2 == 0)
    def _copy():
        k_out[...] = kx_ref[...]
        v_out[...] = vx_ref[...]

    @pl.when(j % 2 == 1)
    def _zero():
        k_out[...] = jnp.zeros_like(k_out)
        v_out[...] = jnp.zeros_like(v_out)


def kernel(kx, vx):
    in_spec = pl.BlockSpec(
        (BATCH, SEQ_BLOCK, KV_HEAD_DIM),
        lambda j: (0, j // 2, 0),
    )
    out_spec = pl.BlockSpec(
        (BATCH, SEQ_BLOCK, KV_HEAD_DIM),
        lambda j: (0, jnp.where(j % 2 == 0, j // 2, N_PREFILL_BLOCKS + j // 2), 0),
    )
    out_shape = jax.ShapeDtypeStruct((BATCH, MAX_SEQ_LEN, KV_HEAD_DIM), jnp.float32)
    return pl.pallas_call(
        _body,
        grid=(N_BLOCKS,),
        in_specs=[in_spec, in_spec],
        out_specs=[out_spec, out_spec],
        out_shape=[out_shape, out_shape],
    )(kx, vx)


# speedup vs baseline: 1.1015x; 1.1015x over previous
import jax
import jax.numpy as jnp
from jax.experimental import pallas as pl

BATCH = 32
MAX_SEQ_LEN = 4096
KV_HEAD_DIM = 128
PREFILL_LEN = 2048

BATCH_BLOCK = 4
N_BLOCKS = BATCH // BATCH_BLOCK


def _body(kx_ref, vx_ref, k_out, v_out):
    zeros = jnp.zeros(
        (BATCH_BLOCK, MAX_SEQ_LEN - PREFILL_LEN, KV_HEAD_DIM), jnp.float32
    )
    k_out[:, :PREFILL_LEN, :] = kx_ref[...]
    k_out[:, PREFILL_LEN:, :] = zeros
    v_out[:, :PREFILL_LEN, :] = vx_ref[...]
    v_out[:, PREFILL_LEN:, :] = zeros


def kernel(kx, vx):
    in_spec = pl.BlockSpec(
        (BATCH_BLOCK, PREFILL_LEN, KV_HEAD_DIM),
        lambda j: (j, 0, 0),
    )
    out_spec = pl.BlockSpec(
        (BATCH_BLOCK, MAX_SEQ_LEN, KV_HEAD_DIM),
        lambda j: (j, 0, 0),
    )
    out_shape = jax.ShapeDtypeStruct((BATCH, MAX_SEQ_LEN, KV_HEAD_DIM), jnp.float32)
    return pl.pallas_call(
        _body,
        grid=(N_BLOCKS,),
        in_specs=[in_spec, in_spec],
        out_specs=[out_spec, out_spec],
        out_shape=[out_shape, out_shape],
    )(kx, vx)
